# baseline (speedup 1.0000x reference)
import functools

import jax
import jax.numpy as jnp
from jax import lax
from jax.experimental import pallas as pl
from jax.experimental.pallas import tpu as pltpu
from jax.experimental.pallas import tpu_sc as plsc

N_SUBCORES = 16
N_CORES = 2
CHUNK = 128
SC_SPLIT = (32, 18)


def _make_sc_agg(n_pad, cpts, h):
    stripe = n_pad // N_SUBCORES
    cmax = max(cpts)
    mesh = plsc.VectorSubcoreMesh(core_axis_name="c", subcore_axis_name="s")

    @functools.partial(
        pl.kernel,
        mesh=mesh,
        compiler_params=pltpu.CompilerParams(use_tc_tiling_on_sc=False),
        out_type=jax.ShapeDtypeStruct((N_CORES, n_pad, h), jnp.float32),
        scratch_types=[
            pltpu.VMEM((cmax, CHUNK), jnp.int32),
            pltpu.VMEM((cmax, CHUNK), jnp.int32),
            pltpu.VMEM((2, CHUNK, h), jnp.float32),
            pltpu.VMEM_SHARED((n_pad, h), jnp.float32),
            pltpu.SemaphoreType.DMA,
            pltpu.SemaphoreType.DMA,
            pltpu.SemaphoreType.DMA,
            pltpu.SemaphoreType.DMA,
        ],
    )
    def sc_agg(y_hbm, idx0_hbm, dst0_hbm, idx1_hbm, dst1_hbm, zeros_hbm,
               out_hbm, idx_v, dst_v, rows_c, agg_sh,
               sem_g0, sem_g1, sem_s0, sem_s1):
        cid = lax.axis_index("c")
        sid = lax.axis_index("s")
        base = sid * stripe
        pltpu.sync_copy(zeros_hbm, agg_sh.at[pl.ds(base, stripe)])
        plsc.subcore_barrier()

        sem_g = (sem_g0, sem_g1)
        sem_s = (sem_s0, sem_s1)

        def run(idx_hbm, dst_hbm, cpt):
            cp_i = pltpu.async_copy(
                idx_hbm.at[sid], idx_v.at[pl.ds(0, cpt)], sem_g0)
            cp_d = pltpu.async_copy(
                dst_hbm.at[sid], dst_v.at[pl.ds(0, cpt)], sem_g1)
            cp_i.wait()
            cp_d.wait()

            def start_gather(j, b):
                pltpu.async_copy(
                    y_hbm.at[idx_v.at[j]], rows_c.at[b], sem_g[b])

            def wait_gather(b):
                pltpu.make_async_copy(
                    y_hbm.at[idx_v.at[0]], rows_c.at[0], sem_g[b]).wait()

            def start_scatter(j, b):
                pltpu.async_copy(rows_c.at[b], agg_sh.at[dst_v.at[j]],
                                 sem_s[b], add=True)

            def wait_scatter(b):
                pltpu.make_async_copy(
                    rows_c.at[b], agg_sh.at[pl.ds(0, CHUNK)],
                    sem_s[b]).wait()

            start_gather(0, 0)
            if cpt > 1:
                start_gather(1, 1)
            wait_gather(0)
            start_scatter(0, 0)

            def chunk_step(j, b):
                wait_scatter(1 - b)
                start_gather(jnp.minimum(j + 1, cpt - 1), 1 - b)
                wait_gather(b)
                start_scatter(j, b)

            def pair_body(jj, carry):
                chunk_step(2 * jj + 1, 1)
                chunk_step(2 * jj + 2, 0)
                return carry

            lax.fori_loop(0, (cpt - 1) // 2, pair_body, 0)
            if (cpt - 1) % 2:
                chunk_step(cpt - 1, 1)
            wait_scatter((cpt - 1) % 2)
            wait_gather(cpt % 2)

        @pl.when(cid == 0)
        def _():
            run(idx0_hbm, dst0_hbm, cpts[0])

        @pl.when(cid == 1)
        def _():
            run(idx1_hbm, dst1_hbm, cpts[1])

        plsc.subcore_barrier()
        pltpu.sync_copy(agg_sh.at[pl.ds(base, stripe)],
                        out_hbm.at[cid, pl.ds(base, stripe)])

    return sc_agg


def _dense_pre(feature, wall, n, h, th, grid, r):
    def body(f_ref, w_ref, y_ref):
        y_ref[:] = jnp.dot(f_ref[:], w_ref[:],
                           preferred_element_type=jnp.float32)

    return pl.pallas_call(
        body,
        grid=(grid,),
        in_specs=[
            pl.BlockSpec((r, h), lambda i: (i, 0)),
            pl.BlockSpec((h, th), lambda i: (0, 0)),
        ],
        out_specs=pl.BlockSpec((r, th), lambda i: (i, 0)),
        out_shape=jax.ShapeDtypeStruct((n, th), jnp.float32),
    )(feature, wall)


def _dense_mid(out_prev, partial, wall, n, n_pad, h, th, grid, r):
    def body(o_ref, p_ref, w_ref, onew_ref, y_ref):
        o = (o_ref[:] + p_ref[0] + p_ref[1]) * 0.5
        onew_ref[:] = o
        y_ref[:] = jnp.dot(o, w_ref[:], preferred_element_type=jnp.float32)

    return pl.pallas_call(
        body,
        grid=(grid,),
        in_specs=[
            pl.BlockSpec((r, h), lambda i: (i, 0)),
            pl.BlockSpec((N_CORES, r, h), lambda i: (0, i, 0)),
            pl.BlockSpec((h, th), lambda i: (0, 0)),
        ],
        out_specs=[
            pl.BlockSpec((r, h), lambda i: (i, 0)),
            pl.BlockSpec((r, th), lambda i: (i, 0)),
        ],
        out_shape=[
            jax.ShapeDtypeStruct((n, h), jnp.float32),
            jax.ShapeDtypeStruct((n, th), jnp.float32),
        ],
    )(out_prev, partial, wall)


def _finish(out_prev, partial, w1, b1, w2, b2, w3, b3, n, h, grid, r):
    d1 = w1.shape[1]
    d2 = w2.shape[1]

    def body(o_ref, p_ref, w1_ref, b1_ref, w2_ref, b2_ref, w3_ref, b3_ref,
             prob_ref):
        o = (o_ref[:] + p_ref[0] + p_ref[1]) * 0.5
        nrm = jnp.sqrt(jnp.sum(o * o, axis=1, keepdims=True))
        o = o / jnp.maximum(nrm, 1e-12)
        hh = jnp.tanh(jnp.dot(o, w1_ref[:],
                              preferred_element_type=jnp.float32) + b1_ref[:])
        hh = jnp.tanh(jnp.dot(hh, w2_ref[:],
                              preferred_element_type=jnp.float32) + b2_ref[:])
        z = jnp.dot(hh, w3_ref[:], preferred_element_type=jnp.float32) \
            + b3_ref[:]
        prob_ref[:] = jax.nn.sigmoid(z)

    return pl.pallas_call(
        body,
        grid=(grid,),
        in_specs=[
            pl.BlockSpec((r, h), lambda i: (i, 0)),
            pl.BlockSpec((N_CORES, r, h), lambda i: (0, i, 0)),
            pl.BlockSpec((h, d1), lambda i: (0, 0)),
            pl.BlockSpec((1, d1), lambda i: (0, 0)),
            pl.BlockSpec((d1, d2), lambda i: (0, 0)),
            pl.BlockSpec((1, d2), lambda i: (0, 0)),
            pl.BlockSpec((d2, 1), lambda i: (0, 0)),
            pl.BlockSpec((1, 1), lambda i: (0, 0)),
        ],
        out_specs=pl.BlockSpec((r, 1), lambda i: (i, 0)),
        out_shape=jax.ShapeDtypeStruct((n, 1), jnp.float32),
    )(out_prev, partial, w1, b1.reshape(1, d1), w2, b2.reshape(1, d2),
      w3, b3.reshape(1, 1))


def kernel(feature, edge_index, param_id, param, W1, b1, W2, b2, W3, b3):
    n, h = feature.shape
    e = edge_index.shape[1]
    t = param.shape[0]
    th = t * h

    cpt0, cpt1 = SC_SPLIT
    e0 = N_SUBCORES * cpt0 * CHUNK
    e_pad = e0 + N_SUBCORES * cpt1 * CHUNK
    assert e_pad >= e
    n_pad = -(-(n + 1) // (16 * N_SUBCORES)) * (16 * N_SUBCORES)
    stripe = n_pad // N_SUBCORES

    grid = 5
    r = n // grid

    src = edge_index[0]
    dst = edge_index[1]
    gidx = src * t + param_id
    pad = e_pad - e
    gidx_p = jnp.concatenate([gidx, jnp.zeros((pad,), jnp.int32)])
    dst_p = jnp.concatenate([dst, jnp.full((pad,), n, jnp.int32)])
    gidx0 = gidx_p[:e0].reshape(N_SUBCORES, cpt0, CHUNK)
    dst0 = dst_p[:e0].reshape(N_SUBCORES, cpt0, CHUNK)
    gidx1 = gidx_p[e0:].reshape(N_SUBCORES, cpt1, CHUNK)
    dst1 = dst_p[e0:].reshape(N_SUBCORES, cpt1, CHUNK)
    zeros_blk = jnp.zeros((stripe, h), jnp.float32)

    wall = param.transpose(1, 0, 2).reshape(h, th)

    sc_agg = _make_sc_agg(n_pad, (cpt0, cpt1), h)

    out = feature
    y = _dense_pre(feature, wall, n, h, th, grid, r)
    for layer in range(3):
        partial = sc_agg(y.reshape(n * t, h), gidx0, dst0, gidx1, dst1,
                         zeros_blk)
        if layer < 2:
            out, y = _dense_mid(out, partial, wall, n, n_pad, h, th, grid, r)
        else:
            probs = _finish(out, partial, W1, b1, W2, b2, W3, b3,
                            n, h, grid, r)
    return probs[:, 0]


# speedup vs baseline: 1.1087x; 1.1087x over previous
import functools

import jax
import jax.numpy as jnp
from jax import lax
from jax.experimental import pallas as pl
from jax.experimental.pallas import tpu as pltpu
from jax.experimental.pallas import tpu_sc as plsc

N_SUBCORES = 16
N_CORES = 2
CHUNK = 128
SC_SPLIT = (31, 19)


def _make_sc_agg(n_pad, cpts, h):
    stripe = n_pad // N_SUBCORES
    cmax = max(cpts)
    mesh = plsc.VectorSubcoreMesh(core_axis_name="c", subcore_axis_name="s")

    @functools.partial(
        pl.kernel,
        mesh=mesh,
        compiler_params=pltpu.CompilerParams(
            use_tc_tiling_on_sc=False, needs_layout_passes=False),
        out_type=jax.ShapeDtypeStruct((N_CORES, n_pad, h), jnp.float32),
        scratch_types=[
            pltpu.VMEM((cmax, CHUNK), jnp.int32),
            pltpu.VMEM((cmax, CHUNK), jnp.int32),
            pltpu.VMEM((2, CHUNK, h), jnp.bfloat16),
            pltpu.VMEM((2, CHUNK, h), jnp.float32),
            pltpu.VMEM_SHARED((n_pad, h), jnp.float32),
            pltpu.SemaphoreType.DMA,
            pltpu.SemaphoreType.DMA,
            pltpu.SemaphoreType.DMA,
            pltpu.SemaphoreType.DMA,
        ],
    )
    def sc_agg(y_hbm, idx0_hbm, dst0_hbm, idx1_hbm, dst1_hbm, zeros_hbm,
               out_hbm, idx_v, dst_v, rows_bf, rows_f, agg_sh,
               sem_g0, sem_g1, sem_s0, sem_s1):
        cid = lax.axis_index("c")
        sid = lax.axis_index("s")
        base = sid * stripe
        pltpu.sync_copy(zeros_hbm, agg_sh.at[pl.ds(base, stripe)])
        plsc.subcore_barrier()

        sem_g = (sem_g0, sem_g1)
        sem_s = (sem_s0, sem_s1)

        def run(idx_hbm, dst_hbm, cpt):
            cp_i = pltpu.async_copy(
                idx_hbm.at[sid], idx_v.at[pl.ds(0, cpt)], sem_g0)
            cp_d = pltpu.async_copy(
                dst_hbm.at[sid], dst_v.at[pl.ds(0, cpt)], sem_g1)
            cp_i.wait()
            cp_d.wait()

            def start_gather(j, b):
                pltpu.async_copy(
                    y_hbm.at[idx_v.at[j]], rows_bf.at[b], sem_g[b])

            def wait_gather(b):
                pltpu.make_async_copy(
                    y_hbm.at[idx_v.at[0]], rows_bf.at[0], sem_g[b]).wait()

            def start_scatter(j, b):
                pltpu.async_copy(rows_f.at[b], agg_sh.at[dst_v.at[j]],
                                 sem_s[b], add=True)

            def wait_scatter(b):
                pltpu.make_async_copy(
                    rows_f.at[b], agg_sh.at[pl.ds(0, CHUNK)],
                    sem_s[b]).wait()

            def convert(b):
                def cbody(i, carry):
                    lo, hi = plsc.unpack(
                        rows_bf[b, i, :], format=plsc.PackFormat.INTERLEAVED)
                    rows_f[b, i, 0:16] = lo
                    rows_f[b, i, 16:32] = hi
                    return carry

                lax.fori_loop(0, CHUNK, cbody, 0)

            start_gather(0, 0)
            if cpt > 1:
                start_gather(1, 1)
            wait_gather(0)
            convert(0)
            start_scatter(0, 0)
            if cpt > 1:
                start_gather(jnp.minimum(2, cpt - 1), 0)
                wait_gather(1)
                convert(1)
                start_scatter(1, 1)

            def chunk_step(j, b):
                start_gather(jnp.minimum(j + 1, cpt - 1), 1 - b)
                wait_scatter(b)
                wait_gather(b)
                convert(b)
                start_scatter(j, b)

            def pair_body(jj, carry):
                chunk_step(2 * jj + 2, 0)
                chunk_step(2 * jj + 3, 1)
                return carry

            if cpt > 2:
                lax.fori_loop(0, (cpt - 2) // 2, pair_body, 0)
                if (cpt - 2) % 2:
                    chunk_step(cpt - 1, (cpt - 1) % 2)
            if cpt > 1:
                wait_scatter((cpt - 2) % 2)
            wait_scatter((cpt - 1) % 2)
            wait_gather(cpt % 2)

        @pl.when(cid == 0)
        def _():
            run(idx0_hbm, dst0_hbm, cpts[0])

        @pl.when(cid == 1)
        def _():
            run(idx1_hbm, dst1_hbm, cpts[1])

        plsc.subcore_barrier()
        pltpu.sync_copy(agg_sh.at[pl.ds(base, stripe)],
                        out_hbm.at[cid, pl.ds(base, stripe)])

    return sc_agg


def _dense_pre(feature, wall, n, h, th, grid, r):
    def body(f_ref, w_ref, y_ref):
        y_ref[:] = jnp.dot(f_ref[:], w_ref[:],
                           preferred_element_type=jnp.float32
                           ).astype(jnp.bfloat16)

    return pl.pallas_call(
        body,
        grid=(grid,),
        in_specs=[
            pl.BlockSpec((r, h), lambda i: (i, 0)),
            pl.BlockSpec((h, th), lambda i: (0, 0)),
        ],
        out_specs=pl.BlockSpec((r, th), lambda i: (i, 0)),
        out_shape=jax.ShapeDtypeStruct((n, th), jnp.bfloat16),
    )(feature, wall)


def _dense_mid(out_prev, partial, wall, n, n_pad, h, th, grid, r):
    def body(o_ref, p_ref, w_ref, onew_ref, y_ref):
        o = (o_ref[:] + p_ref[0] + p_ref[1]) * 0.5
        onew_ref[:] = o
        y_ref[:] = jnp.dot(o, w_ref[:], preferred_element_type=jnp.float32
                           ).astype(jnp.bfloat16)

    return pl.pallas_call(
        body,
        grid=(grid,),
        in_specs=[
            pl.BlockSpec((r, h), lambda i: (i, 0)),
            pl.BlockSpec((N_CORES, r, h), lambda i: (0, i, 0)),
            pl.BlockSpec((h, th), lambda i: (0, 0)),
        ],
        out_specs=[
            pl.BlockSpec((r, h), lambda i: (i, 0)),
            pl.BlockSpec((r, th), lambda i: (i, 0)),
        ],
        out_shape=[
            jax.ShapeDtypeStruct((n, h), jnp.float32),
            jax.ShapeDtypeStruct((n, th), jnp.bfloat16),
        ],
    )(out_prev, partial, wall)


def _finish(out_prev, partial, w1, b1, w2, b2, w3, b3, n, h, grid, r):
    d1 = w1.shape[1]
    d2 = w2.shape[1]

    def body(o_ref, p_ref, w1_ref, b1_ref, w2_ref, b2_ref, w3_ref, b3_ref,
             prob_ref):
        o = (o_ref[:] + p_ref[0] + p_ref[1]) * 0.5
        nrm = jnp.sqrt(jnp.sum(o * o, axis=1, keepdims=True))
        o = o / jnp.maximum(nrm, 1e-12)
        hh = jnp.tanh(jnp.dot(o, w1_ref[:],
                              preferred_element_type=jnp.float32) + b1_ref[:])
        hh = jnp.tanh(jnp.dot(hh, w2_ref[:],
                              preferred_element_type=jnp.float32) + b2_ref[:])
        z = jnp.dot(hh, w3_ref[:], preferred_element_type=jnp.float32) \
            + b3_ref[:]
        prob_ref[:] = jax.nn.sigmoid(z)

    return pl.pallas_call(
        body,
        grid=(grid,),
        in_specs=[
            pl.BlockSpec((r, h), lambda i: (i, 0)),
            pl.BlockSpec((N_CORES, r, h), lambda i: (0, i, 0)),
            pl.BlockSpec((h, d1), lambda i: (0, 0)),
            pl.BlockSpec((1, d1), lambda i: (0, 0)),
            pl.BlockSpec((d1, d2), lambda i: (0, 0)),
            pl.BlockSpec((1, d2), lambda i: (0, 0)),
            pl.BlockSpec((d2, 1), lambda i: (0, 0)),
            pl.BlockSpec((1, 1), lambda i: (0, 0)),
        ],
        out_specs=pl.BlockSpec((r, 1), lambda i: (i, 0)),
        out_shape=jax.ShapeDtypeStruct((n, 1), jnp.float32),
    )(out_prev, partial, w1, b1.reshape(1, d1), w2, b2.reshape(1, d2),
      w3, b3.reshape(1, 1))


def kernel(feature, edge_index, param_id, param, W1, b1, W2, b2, W3, b3):
    n, h = feature.shape
    e = edge_index.shape[1]
    t = param.shape[0]
    th = t * h

    cpt0, cpt1 = SC_SPLIT
    e0 = N_SUBCORES * cpt0 * CHUNK
    e_pad = e0 + N_SUBCORES * cpt1 * CHUNK
    assert e_pad >= e
    n_pad = -(-(n + 1) // (16 * N_SUBCORES)) * (16 * N_SUBCORES)
    stripe = n_pad // N_SUBCORES

    grid = 5
    r = n // grid

    src = edge_index[0]
    dst = edge_index[1]
    gidx = src * t + param_id
    pad = e_pad - e
    gidx_p = jnp.concatenate([gidx, jnp.zeros((pad,), jnp.int32)])
    dst_p = jnp.concatenate([dst, jnp.full((pad,), n, jnp.int32)])
    gidx0 = gidx_p[:e0].reshape(N_SUBCORES, cpt0, CHUNK)
    dst0 = dst_p[:e0].reshape(N_SUBCORES, cpt0, CHUNK)
    gidx1 = gidx_p[e0:].reshape(N_SUBCORES, cpt1, CHUNK)
    dst1 = dst_p[e0:].reshape(N_SUBCORES, cpt1, CHUNK)
    zeros_blk = jnp.zeros((stripe, h), jnp.float32)

    wall = param.transpose(1, 0, 2).reshape(h, th)
    k = jnp.arange(th)
    within = k % h
    src_col = (k // h) * h + (within % 2) * (h // 2) + within // 2
    wall = wall[:, src_col]

    sc_agg = _make_sc_agg(n_pad, (cpt0, cpt1), h)

    out = feature
    y = _dense_pre(feature, wall, n, h, th, grid, r)
    for layer in range(3):
        partial = sc_agg(y.reshape(n * t, h), gidx0, dst0, gidx1, dst1,
                         zeros_blk)
        if layer < 2:
            out, y = _dense_mid(out, partial, wall, n, n_pad, h, th, grid, r)
        else:
            probs = _finish(out, partial, W1, b1, W2, b2, W3, b3,
                            n, h, grid, r)
    return probs[:, 0]


# speedup vs baseline: 1.1364x; 1.0250x over previous
import functools

import jax
import jax.numpy as jnp
from jax import lax
from jax.experimental import pallas as pl
from jax.experimental.pallas import tpu as pltpu
from jax.experimental.pallas import tpu_sc as plsc

N_SUBCORES = 16
N_CORES = 2
CHUNK = 128
SC_SPLIT = (31, 19)


def _make_sc_agg(n_pad, cpts, h):
    stripe = n_pad // N_SUBCORES
    cmax = max(cpts)
    mesh = plsc.VectorSubcoreMesh(core_axis_name="c", subcore_axis_name="s")

    @functools.partial(
        pl.kernel,
        mesh=mesh,
        compiler_params=pltpu.CompilerParams(
            use_tc_tiling_on_sc=False, needs_layout_passes=False),
        out_type=jax.ShapeDtypeStruct((N_CORES, n_pad, h), jnp.float32),
        scratch_types=[
            pltpu.VMEM((cmax, CHUNK), jnp.int32),
            pltpu.VMEM((cmax, CHUNK), jnp.int32),
            pltpu.VMEM((2, CHUNK, h), jnp.bfloat16),
            pltpu.VMEM((2, CHUNK, h), jnp.float32),
            pltpu.VMEM_SHARED((n_pad, h), jnp.float32),
            pltpu.SemaphoreType.DMA,
            pltpu.SemaphoreType.DMA,
            pltpu.SemaphoreType.DMA,
            pltpu.SemaphoreType.DMA,
        ],
    )
    def sc_agg(y_hbm, idx0_hbm, dst0_hbm, idx1_hbm, dst1_hbm, zeros_hbm,
               out_hbm, idx_v, dst_v, rows_bf, rows_f, agg_sh,
               sem_g0, sem_g1, sem_s0, sem_s1):
        cid = lax.axis_index("c")
        sid = lax.axis_index("s")
        base = sid * stripe
        pltpu.sync_copy(zeros_hbm, agg_sh.at[pl.ds(base, stripe)])
        plsc.subcore_barrier()

        sem_g = (sem_g0, sem_g1)
        sem_s = (sem_s0, sem_s1)

        def run(idx_hbm, dst_hbm, cpt):
            cp_i = pltpu.async_copy(
                idx_hbm.at[sid], idx_v.at[pl.ds(0, cpt)], sem_g0)
            cp_d = pltpu.async_copy(
                dst_hbm.at[sid], dst_v.at[pl.ds(0, cpt)], sem_g1)
            cp_i.wait()
            cp_d.wait()

            def start_gather(j, b):
                pltpu.async_copy(
                    y_hbm.at[idx_v.at[j]], rows_bf.at[b], sem_g[b])

            def wait_gather(b):
                pltpu.make_async_copy(
                    y_hbm.at[idx_v.at[0]], rows_bf.at[0], sem_g[b]).wait()

            def start_scatter(j, b):
                pltpu.async_copy(rows_f.at[b], agg_sh.at[dst_v.at[j]],
                                 sem_s[b], add=True)

            def wait_scatter(b):
                pltpu.make_async_copy(
                    rows_f.at[b], agg_sh.at[pl.ds(0, CHUNK)],
                    sem_s[b]).wait()

            def convert(b):
                def cbody(i, carry):
                    lo, hi = plsc.unpack(
                        rows_bf[b, i, :], format=plsc.PackFormat.INTERLEAVED)
                    rows_f[b, i, 0:16] = lo
                    rows_f[b, i, 16:32] = hi
                    return carry

                lax.fori_loop(0, CHUNK, cbody, 0)

            start_gather(0, 0)
            if cpt > 1:
                start_gather(1, 1)
            wait_gather(0)
            convert(0)
            start_scatter(0, 0)
            if cpt > 1:
                start_gather(jnp.minimum(2, cpt - 1), 0)
                wait_gather(1)
                convert(1)
                start_scatter(1, 1)

            def chunk_step(j, b):
                start_gather(jnp.minimum(j + 1, cpt - 1), 1 - b)
                wait_scatter(b)
                wait_gather(b)
                convert(b)
                start_scatter(j, b)

            def pair_body(jj, carry):
                chunk_step(2 * jj + 2, 0)
                chunk_step(2 * jj + 3, 1)
                return carry

            if cpt > 2:
                lax.fori_loop(0, (cpt - 2) // 2, pair_body, 0)
                if (cpt - 2) % 2:
                    chunk_step(cpt - 1, (cpt - 1) % 2)
            if cpt > 1:
                wait_scatter((cpt - 2) % 2)
            wait_scatter((cpt - 1) % 2)
            wait_gather(cpt % 2)

        @pl.when(cid == 0)
        def _():
            run(idx0_hbm, dst0_hbm, cpts[0])

        @pl.when(cid == 1)
        def _():
            run(idx1_hbm, dst1_hbm, cpts[1])

        plsc.subcore_barrier()
        pltpu.sync_copy(agg_sh.at[pl.ds(base, stripe)],
                        out_hbm.at[cid, pl.ds(base, stripe)])

    return sc_agg


def _dense_pre(feature, wall, n, h, th, grid, r):
    def body(f_ref, w_ref, y_ref):
        y_ref[:] = jnp.dot(f_ref[:], w_ref[:],
                           preferred_element_type=jnp.float32
                           ).astype(jnp.bfloat16)

    return pl.pallas_call(
        body,
        grid=(grid,),
        in_specs=[
            pl.BlockSpec((r, h), lambda i: (i, 0)),
            pl.BlockSpec((h, th), lambda i: (0, 0)),
        ],
        out_specs=pl.BlockSpec((r, th), lambda i: (i, 0)),
        out_shape=jax.ShapeDtypeStruct((n, th), jnp.bfloat16),
    )(feature, wall)


def _dense_mid(out_prev, partial, wall, n, n_pad, h, th, grid, r):
    def body(o_ref, p_ref, w_ref, onew_ref, y_ref):
        o = (o_ref[:] + p_ref[0] + p_ref[1]) * 0.5
        onew_ref[:] = o
        y_ref[:] = jnp.dot(o, w_ref[:], preferred_element_type=jnp.float32
                           ).astype(jnp.bfloat16)

    return pl.pallas_call(
        body,
        grid=(grid,),
        in_specs=[
            pl.BlockSpec((r, h), lambda i: (i, 0)),
            pl.BlockSpec((N_CORES, r, h), lambda i: (0, i, 0)),
            pl.BlockSpec((h, th), lambda i: (0, 0)),
        ],
        out_specs=[
            pl.BlockSpec((r, h), lambda i: (i, 0)),
            pl.BlockSpec((r, th), lambda i: (i, 0)),
        ],
        out_shape=[
            jax.ShapeDtypeStruct((n, h), jnp.float32),
            jax.ShapeDtypeStruct((n, th), jnp.bfloat16),
        ],
    )(out_prev, partial, wall)


def _finish(out_prev, partial, w1, b1, w2, b2, w3, b3, n, h, grid, r):
    d1 = w1.shape[1]
    d2 = w2.shape[1]

    def body(o_ref, p_ref, w1_ref, b1_ref, w2_ref, b2_ref, w3_ref, b3_ref,
             prob_ref):
        o = (o_ref[:] + p_ref[0] + p_ref[1]) * 0.5
        nrm = jnp.sqrt(jnp.sum(o * o, axis=1, keepdims=True))
        o = o / jnp.maximum(nrm, 1e-12)
        hh = jnp.tanh(jnp.dot(o, w1_ref[:],
                              preferred_element_type=jnp.float32) + b1_ref[:])
        hh = jnp.tanh(jnp.dot(hh, w2_ref[:],
                              preferred_element_type=jnp.float32) + b2_ref[:])
        z = jnp.dot(hh, w3_ref[:], preferred_element_type=jnp.float32) \
            + b3_ref[:]
        prob_ref[:] = jax.nn.sigmoid(z[:, 0])

    return pl.pallas_call(
        body,
        grid=(grid,),
        in_specs=[
            pl.BlockSpec((r, h), lambda i: (i, 0)),
            pl.BlockSpec((N_CORES, r, h), lambda i: (0, i, 0)),
            pl.BlockSpec((h, d1), lambda i: (0, 0)),
            pl.BlockSpec((1, d1), lambda i: (0, 0)),
            pl.BlockSpec((d1, d2), lambda i: (0, 0)),
            pl.BlockSpec((1, d2), lambda i: (0, 0)),
            pl.BlockSpec((d2, 1), lambda i: (0, 0)),
            pl.BlockSpec((1, 1), lambda i: (0, 0)),
        ],
        out_specs=pl.BlockSpec((r,), lambda i: (i,)),
        out_shape=jax.ShapeDtypeStruct((n,), jnp.float32),
    )(out_prev, partial, w1, b1.reshape(1, d1), w2, b2.reshape(1, d2),
      w3, b3.reshape(1, 1))


def kernel(feature, edge_index, param_id, param, W1, b1, W2, b2, W3, b3):
    n, h = feature.shape
    e = edge_index.shape[1]
    t = param.shape[0]
    th = t * h

    cpt0, cpt1 = SC_SPLIT
    e0 = N_SUBCORES * cpt0 * CHUNK
    e_pad = e0 + N_SUBCORES * cpt1 * CHUNK
    assert e_pad >= e
    n_pad = -(-(n + 1) // (16 * N_SUBCORES)) * (16 * N_SUBCORES)
    stripe = n_pad // N_SUBCORES

    grid = 1
    r = n // grid

    src = edge_index[0]
    dst = edge_index[1]
    gidx = src * t + param_id
    pad = e_pad - e
    gidx_p = jnp.concatenate([gidx, jnp.zeros((pad,), jnp.int32)])
    dst_p = jnp.concatenate([dst, jnp.full((pad,), n, jnp.int32)])
    gidx0 = gidx_p[:e0].reshape(N_SUBCORES, cpt0, CHUNK)
    dst0 = dst_p[:e0].reshape(N_SUBCORES, cpt0, CHUNK)
    gidx1 = gidx_p[e0:].reshape(N_SUBCORES, cpt1, CHUNK)
    dst1 = dst_p[e0:].reshape(N_SUBCORES, cpt1, CHUNK)
    zeros_blk = jnp.zeros((stripe, h), jnp.float32)

    wall = param.transpose(1, 0, 2).reshape(h, th)
    k = jnp.arange(th)
    within = k % h
    src_col = (k // h) * h + (within % 2) * (h // 2) + within // 2
    wall = wall[:, src_col]

    sc_agg = _make_sc_agg(n_pad, (cpt0, cpt1), h)

    out = feature
    y = _dense_pre(feature, wall, n, h, th, grid, r)
    for layer in range(3):
        partial = sc_agg(y.reshape(n * t, h), gidx0, dst0, gidx1, dst1,
                         zeros_blk)
        if layer < 2:
            out, y = _dense_mid(out, partial, wall, n, n_pad, h, th, grid, r)
        else:
            probs = _finish(out, partial, W1, b1, W2, b2, W3, b3,
                            n, h, grid, r)
    return probs


# speedup vs baseline: 1.1612x; 1.0218x over previous
import functools

import jax
import jax.numpy as jnp
from jax import lax
from jax.experimental import pallas as pl
from jax.experimental.pallas import tpu as pltpu
from jax.experimental.pallas import tpu_sc as plsc

N_SUBCORES = 16
N_CORES = 2
CHUNK = 128
SC_SPLIT = (29, 21)


def _make_sc_agg(n_pad, cpts, h):
    stripe = n_pad // N_SUBCORES
    cmax = max(cpts)
    mesh = plsc.VectorSubcoreMesh(core_axis_name="c", subcore_axis_name="s")

    @functools.partial(
        pl.kernel,
        mesh=mesh,
        compiler_params=pltpu.CompilerParams(
            use_tc_tiling_on_sc=False, needs_layout_passes=False),
        out_type=jax.ShapeDtypeStruct((N_CORES, n_pad, h), jnp.float32),
        scratch_types=[
            pltpu.VMEM((cmax, CHUNK), jnp.int32),
            pltpu.VMEM((cmax, CHUNK), jnp.int32),
            pltpu.VMEM((2, CHUNK, h), jnp.bfloat16),
            pltpu.VMEM((2, CHUNK, h), jnp.float32),
            pltpu.VMEM_SHARED((n_pad, h), jnp.float32),
            pltpu.SemaphoreType.DMA,
            pltpu.SemaphoreType.DMA,
            pltpu.SemaphoreType.DMA,
            pltpu.SemaphoreType.DMA,
        ],
    )
    def sc_agg(y_hbm, idx0_hbm, dst0_hbm, idx1_hbm, dst1_hbm, zeros_hbm,
               out_hbm, idx_v, dst_v, rows_bf, rows_f, agg_sh,
               sem_g0, sem_g1, sem_s0, sem_s1):
        cid = lax.axis_index("c")
        sid = lax.axis_index("s")
        base = sid * stripe
        pltpu.sync_copy(zeros_hbm, agg_sh.at[pl.ds(base, stripe)])
        plsc.subcore_barrier()

        sem_g = (sem_g0, sem_g1)
        sem_s = (sem_s0, sem_s1)

        def run(idx_hbm, dst_hbm, cpt):
            cp_i = pltpu.async_copy(
                idx_hbm.at[sid], idx_v.at[pl.ds(0, cpt)], sem_g0)
            cp_d = pltpu.async_copy(
                dst_hbm.at[sid], dst_v.at[pl.ds(0, cpt)], sem_g1)
            cp_i.wait()
            cp_d.wait()

            def start_gather(j, b):
                pltpu.async_copy(
                    y_hbm.at[idx_v.at[j]], rows_bf.at[b], sem_g[b])

            def wait_gather(b):
                pltpu.make_async_copy(
                    y_hbm.at[idx_v.at[0]], rows_bf.at[0], sem_g[b]).wait()

            def start_scatter(j, b):
                pltpu.async_copy(rows_f.at[b], agg_sh.at[dst_v.at[j]],
                                 sem_s[b], add=True)

            def wait_scatter(b):
                pltpu.make_async_copy(
                    rows_f.at[b], agg_sh.at[pl.ds(0, CHUNK)],
                    sem_s[b]).wait()

            def convert(b):
                def cbody(i, carry):
                    lo, hi = plsc.unpack(
                        rows_bf[b, i, :], format=plsc.PackFormat.INTERLEAVED)
                    rows_f[b, i, 0:16] = lo
                    rows_f[b, i, 16:32] = hi
                    return carry

                lax.fori_loop(0, CHUNK, cbody, 0)

            start_gather(0, 0)
            if cpt > 1:
                start_gather(1, 1)
            wait_gather(0)
            convert(0)
            start_scatter(0, 0)
            if cpt > 1:
                start_gather(jnp.minimum(2, cpt - 1), 0)
                wait_gather(1)
                convert(1)
                start_scatter(1, 1)

            def chunk_step(j, b):
                start_gather(jnp.minimum(j + 1, cpt - 1), 1 - b)
                wait_scatter(b)
                wait_gather(b)
                convert(b)
                start_scatter(j, b)

            def pair_body(jj, carry):
                chunk_step(2 * jj + 2, 0)
                chunk_step(2 * jj + 3, 1)
                return carry

            if cpt > 2:
                lax.fori_loop(0, (cpt - 2) // 2, pair_body, 0)
                if (cpt - 2) % 2:
                    chunk_step(cpt - 1, (cpt - 1) % 2)
            if cpt > 1:
                wait_scatter((cpt - 2) % 2)
            wait_scatter((cpt - 1) % 2)
            wait_gather(cpt % 2)

        @pl.when(cid == 0)
        def _():
            run(idx0_hbm, dst0_hbm, cpts[0])

        @pl.when(cid == 1)
        def _():
            run(idx1_hbm, dst1_hbm, cpts[1])

        plsc.subcore_barrier()
        pltpu.sync_copy(agg_sh.at[pl.ds(base, stripe)],
                        out_hbm.at[cid, pl.ds(base, stripe)])

    return sc_agg


def _dense_pre(feature, wall, n, h, th, grid, r):
    def body(f_ref, w_ref, y_ref):
        y_ref[:] = jnp.dot(f_ref[:], w_ref[:],
                           preferred_element_type=jnp.float32
                           ).astype(jnp.bfloat16)

    return pl.pallas_call(
        body,
        grid=(grid,),
        in_specs=[
            pl.BlockSpec((r, h), lambda i: (i, 0)),
            pl.BlockSpec((h, th), lambda i: (0, 0)),
        ],
        out_specs=pl.BlockSpec((r, th), lambda i: (i, 0)),
        out_shape=jax.ShapeDtypeStruct((n, th), jnp.bfloat16),
    )(feature, wall)


def _dense_mid(out_prev, partial, wall, n, n_pad, h, th, grid, r):
    def body(o_ref, p_ref, w_ref, onew_ref, y_ref):
        o = (o_ref[:] + p_ref[0] + p_ref[1]) * 0.5
        onew_ref[:] = o
        y_ref[:] = jnp.dot(o, w_ref[:], preferred_element_type=jnp.float32
                           ).astype(jnp.bfloat16)

    return pl.pallas_call(
        body,
        grid=(grid,),
        in_specs=[
            pl.BlockSpec((r, h), lambda i: (i, 0)),
            pl.BlockSpec((N_CORES, r, h), lambda i: (0, i, 0)),
            pl.BlockSpec((h, th), lambda i: (0, 0)),
        ],
        out_specs=[
            pl.BlockSpec((r, h), lambda i: (i, 0)),
            pl.BlockSpec((r, th), lambda i: (i, 0)),
        ],
        out_shape=[
            jax.ShapeDtypeStruct((n, h), jnp.float32),
            jax.ShapeDtypeStruct((n, th), jnp.bfloat16),
        ],
    )(out_prev, partial, wall)


def _finish(out_prev, partial, w1, b1, w2, b2, w3, b3, n, h, grid, r):
    d1 = w1.shape[1]
    d2 = w2.shape[1]

    def body(o_ref, p_ref, w1_ref, b1_ref, w2_ref, b2_ref, w3_ref, b3_ref,
             prob_ref):
        o = (o_ref[:] + p_ref[0] + p_ref[1]) * 0.5
        nrm = jnp.sqrt(jnp.sum(o * o, axis=1, keepdims=True))
        o = o / jnp.maximum(nrm, 1e-12)
        hh = jnp.tanh(jnp.dot(o, w1_ref[:],
                              preferred_element_type=jnp.float32) + b1_ref[:])
        hh = jnp.tanh(jnp.dot(hh, w2_ref[:],
                              preferred_element_type=jnp.float32) + b2_ref[:])
        z = jnp.dot(hh, w3_ref[:], preferred_element_type=jnp.float32) \
            + b3_ref[:]
        prob_ref[:] = jax.nn.sigmoid(z[:, 0])

    return pl.pallas_call(
        body,
        grid=(grid,),
        in_specs=[
            pl.BlockSpec((r, h), lambda i: (i, 0)),
            pl.BlockSpec((N_CORES, r, h), lambda i: (0, i, 0)),
            pl.BlockSpec((h, d1), lambda i: (0, 0)),
            pl.BlockSpec((1, d1), lambda i: (0, 0)),
            pl.BlockSpec((d1, d2), lambda i: (0, 0)),
            pl.BlockSpec((1, d2), lambda i: (0, 0)),
            pl.BlockSpec((d2, 1), lambda i: (0, 0)),
            pl.BlockSpec((1, 1), lambda i: (0, 0)),
        ],
        out_specs=pl.BlockSpec((r,), lambda i: (i,)),
        out_shape=jax.ShapeDtypeStruct((n,), jnp.float32),
    )(out_prev, partial, w1, b1.reshape(1, d1), w2, b2.reshape(1, d2),
      w3, b3.reshape(1, 1))


def kernel(feature, edge_index, param_id, param, W1, b1, W2, b2, W3, b3):
    n, h = feature.shape
    e = edge_index.shape[1]
    t = param.shape[0]
    th = t * h

    cpt0, cpt1 = SC_SPLIT
    e0 = N_SUBCORES * cpt0 * CHUNK
    e_pad = e0 + N_SUBCORES * cpt1 * CHUNK
    assert e_pad >= e
    n_pad = -(-(n + 1) // (16 * N_SUBCORES)) * (16 * N_SUBCORES)
    stripe = n_pad // N_SUBCORES

    grid = 1
    r = n // grid

    src = edge_index[0]
    dst = edge_index[1]
    gidx = src * t + param_id
    pad = e_pad - e
    gidx_p = jnp.concatenate([gidx, jnp.zeros((pad,), jnp.int32)])
    dst_p = jnp.concatenate([dst, jnp.full((pad,), n, jnp.int32)])
    gidx0 = gidx_p[:e0].reshape(N_SUBCORES, cpt0, CHUNK)
    dst0 = dst_p[:e0].reshape(N_SUBCORES, cpt0, CHUNK)
    gidx1 = gidx_p[e0:].reshape(N_SUBCORES, cpt1, CHUNK)
    dst1 = dst_p[e0:].reshape(N_SUBCORES, cpt1, CHUNK)
    zeros_blk = jnp.zeros((stripe, h), jnp.float32)

    wall = param.transpose(1, 0, 2).reshape(h, th)
    k = jnp.arange(th)
    within = k % h
    src_col = (k // h) * h + (within % 2) * (h // 2) + within // 2
    wall = wall[:, src_col]

    sc_agg = _make_sc_agg(n_pad, (cpt0, cpt1), h)

    out = feature
    y = _dense_pre(feature, wall, n, h, th, grid, r)
    for layer in range(3):
        partial = sc_agg(y.reshape(n * t, h), gidx0, dst0, gidx1, dst1,
                         zeros_blk)
        if layer < 2:
            out, y = _dense_mid(out, partial, wall, n, n_pad, h, th, grid, r)
        else:
            probs = _finish(out, partial, W1, b1, W2, b2, W3, b3,
                            n, h, grid, r)
    return probs


# speedup vs baseline: 1.1641x; 1.0025x over previous
import functools

import jax
import jax.numpy as jnp
from jax import lax
from jax.experimental import pallas as pl
from jax.experimental.pallas import tpu as pltpu
from jax.experimental.pallas import tpu_sc as plsc

N_SUBCORES = 16
N_CORES = 2
CHUNK = 128
SC_SPLIT = (28, 22)


def _make_sc_agg(n_pad, cpts, h):
    stripe = n_pad // N_SUBCORES
    cmax = max(cpts)
    mesh = plsc.VectorSubcoreMesh(core_axis_name="c", subcore_axis_name="s")

    @functools.partial(
        pl.kernel,
        mesh=mesh,
        compiler_params=pltpu.CompilerParams(
            use_tc_tiling_on_sc=False, needs_layout_passes=False),
        out_type=jax.ShapeDtypeStruct((N_CORES, n_pad, h), jnp.float32),
        scratch_types=[
            pltpu.VMEM((cmax, CHUNK), jnp.int32),
            pltpu.VMEM((cmax, CHUNK), jnp.int32),
            pltpu.VMEM((2, CHUNK, h), jnp.bfloat16),
            pltpu.VMEM((2, CHUNK, h), jnp.float32),
            pltpu.VMEM_SHARED((n_pad, h), jnp.float32),
            pltpu.SemaphoreType.DMA,
            pltpu.SemaphoreType.DMA,
            pltpu.SemaphoreType.DMA,
            pltpu.SemaphoreType.DMA,
        ],
    )
    def sc_agg(y_hbm, idx0_hbm, dst0_hbm, idx1_hbm, dst1_hbm, zeros_hbm,
               out_hbm, idx_v, dst_v, rows_bf, rows_f, agg_sh,
               sem_g0, sem_g1, sem_s0, sem_s1):
        cid = lax.axis_index("c")
        sid = lax.axis_index("s")
        base = sid * stripe
        pltpu.sync_copy(zeros_hbm, agg_sh.at[pl.ds(base, stripe)])
        plsc.subcore_barrier()

        sem_g = (sem_g0, sem_g1)
        sem_s = (sem_s0, sem_s1)

        def run(idx_hbm, dst_hbm, cpt):
            cp_i = pltpu.async_copy(
                idx_hbm.at[sid], idx_v.at[pl.ds(0, cpt)], sem_g0)
            cp_d = pltpu.async_copy(
                dst_hbm.at[sid], dst_v.at[pl.ds(0, cpt)], sem_g1)
            cp_i.wait()
            cp_d.wait()

            def start_gather(j, b):
                pltpu.async_copy(
                    y_hbm.at[idx_v.at[j]], rows_bf.at[b], sem_g[b])

            def wait_gather(b):
                pltpu.make_async_copy(
                    y_hbm.at[idx_v.at[0]], rows_bf.at[0], sem_g[b]).wait()

            def start_scatter(j, b):
                pltpu.async_copy(rows_f.at[b], agg_sh.at[dst_v.at[j]],
                                 sem_s[b], add=True)

            def wait_scatter(b):
                pltpu.make_async_copy(
                    rows_f.at[b], agg_sh.at[pl.ds(0, CHUNK)],
                    sem_s[b]).wait()

            def convert(b):
                def cbody(i, carry):
                    lo, hi = plsc.unpack(
                        rows_bf[b, i, :], format=plsc.PackFormat.INTERLEAVED)
                    rows_f[b, i, 0:16] = lo
                    rows_f[b, i, 16:32] = hi
                    return carry

                lax.fori_loop(0, CHUNK, cbody, 0)

            start_gather(0, 0)
            if cpt > 1:
                start_gather(1, 1)
            wait_gather(0)
            convert(0)
            start_scatter(0, 0)
            if cpt > 1:
                start_gather(jnp.minimum(2, cpt - 1), 0)
                wait_gather(1)
                convert(1)
                start_scatter(1, 1)

            def chunk_step(j, b):
                start_gather(jnp.minimum(j + 1, cpt - 1), 1 - b)
                wait_scatter(b)
                wait_gather(b)
                convert(b)
                start_scatter(j, b)

            def pair_body(jj, carry):
                chunk_step(2 * jj + 2, 0)
                chunk_step(2 * jj + 3, 1)
                return carry

            if cpt > 2:
                lax.fori_loop(0, (cpt - 2) // 2, pair_body, 0)
                if (cpt - 2) % 2:
                    chunk_step(cpt - 1, (cpt - 1) % 2)
            if cpt > 1:
                wait_scatter((cpt - 2) % 2)
            wait_scatter((cpt - 1) % 2)
            wait_gather(cpt % 2)

        @pl.when(cid == 0)
        def _():
            run(idx0_hbm, dst0_hbm, cpts[0])

        @pl.when(cid == 1)
        def _():
            run(idx1_hbm, dst1_hbm, cpts[1])

        plsc.subcore_barrier()
        pltpu.sync_copy(agg_sh.at[pl.ds(base, stripe)],
                        out_hbm.at[cid, pl.ds(base, stripe)])

    return sc_agg


def _dense_pre(feature, wall, n, h, th, grid, r):
    def body(f_ref, w_ref, y_ref):
        y_ref[:] = jnp.dot(f_ref[:], w_ref[:],
                           preferred_element_type=jnp.float32
                           ).astype(jnp.bfloat16)

    return pl.pallas_call(
        body,
        grid=(grid,),
        in_specs=[
            pl.BlockSpec((r, h), lambda i: (i, 0)),
            pl.BlockSpec((h, th), lambda i: (0, 0)),
        ],
        out_specs=pl.BlockSpec((r, th), lambda i: (i, 0)),
        out_shape=jax.ShapeDtypeStruct((n, th), jnp.bfloat16),
    )(feature, wall)


def _dense_mid(out_prev, partial, wall, n, n_pad, h, th, grid, r):
    def body(o_ref, p_ref, w_ref, onew_ref, y_ref):
        o = (o_ref[:] + p_ref[0] + p_ref[1]) * 0.5
        onew_ref[:] = o
        y_ref[:] = jnp.dot(o, w_ref[:], preferred_element_type=jnp.float32
                           ).astype(jnp.bfloat16)

    return pl.pallas_call(
        body,
        grid=(grid,),
        in_specs=[
            pl.BlockSpec((r, h), lambda i: (i, 0)),
            pl.BlockSpec((N_CORES, r, h), lambda i: (0, i, 0)),
            pl.BlockSpec((h, th), lambda i: (0, 0)),
        ],
        out_specs=[
            pl.BlockSpec((r, h), lambda i: (i, 0)),
            pl.BlockSpec((r, th), lambda i: (i, 0)),
        ],
        out_shape=[
            jax.ShapeDtypeStruct((n, h), jnp.float32),
            jax.ShapeDtypeStruct((n, th), jnp.bfloat16),
        ],
    )(out_prev, partial, wall)


def _finish(out_prev, partial, w1, b1, w2, b2, w3, b3, n, h, grid, r):
    d1 = w1.shape[1]
    d2 = w2.shape[1]

    def body(o_ref, p_ref, w1_ref, b1_ref, w2_ref, b2_ref, w3_ref, b3_ref,
             prob_ref):
        o = (o_ref[:] + p_ref[0] + p_ref[1]) * 0.5
        nrm = jnp.sqrt(jnp.sum(o * o, axis=1, keepdims=True))
        o = o / jnp.maximum(nrm, 1e-12)
        hh = jnp.tanh(jnp.dot(o, w1_ref[:],
                              preferred_element_type=jnp.float32) + b1_ref[:])
        hh = jnp.tanh(jnp.dot(hh, w2_ref[:],
                              preferred_element_type=jnp.float32) + b2_ref[:])
        z = jnp.dot(hh, w3_ref[:], preferred_element_type=jnp.float32) \
            + b3_ref[:]
        prob_ref[:] = jax.nn.sigmoid(z[:, 0])

    return pl.pallas_call(
        body,
        grid=(grid,),
        in_specs=[
            pl.BlockSpec((r, h), lambda i: (i, 0)),
            pl.BlockSpec((N_CORES, r, h), lambda i: (0, i, 0)),
            pl.BlockSpec((h, d1), lambda i: (0, 0)),
            pl.BlockSpec((1, d1), lambda i: (0, 0)),
            pl.BlockSpec((d1, d2), lambda i: (0, 0)),
            pl.BlockSpec((1, d2), lambda i: (0, 0)),
            pl.BlockSpec((d2, 1), lambda i: (0, 0)),
            pl.BlockSpec((1, 1), lambda i: (0, 0)),
        ],
        out_specs=pl.BlockSpec((r,), lambda i: (i,)),
        out_shape=jax.ShapeDtypeStruct((n,), jnp.float32),
    )(out_prev, partial, w1, b1.reshape(1, d1), w2, b2.reshape(1, d2),
      w3, b3.reshape(1, 1))


def kernel(feature, edge_index, param_id, param, W1, b1, W2, b2, W3, b3):
    n, h = feature.shape
    e = edge_index.shape[1]
    t = param.shape[0]
    th = t * h

    cpt0, cpt1 = SC_SPLIT
    e0 = N_SUBCORES * cpt0 * CHUNK
    e_pad = e0 + N_SUBCORES * cpt1 * CHUNK
    assert e_pad >= e
    n_pad = -(-(n + 1) // (16 * N_SUBCORES)) * (16 * N_SUBCORES)
    stripe = n_pad // N_SUBCORES

    grid = 1
    r = n // grid

    src = edge_index[0]
    dst = edge_index[1]
    gidx = src * t + param_id
    pad = e_pad - e
    gidx_p = jnp.concatenate([gidx, jnp.zeros((pad,), jnp.int32)])
    dst_p = jnp.concatenate([dst, jnp.full((pad,), n, jnp.int32)])
    gidx0 = gidx_p[:e0].reshape(N_SUBCORES, cpt0, CHUNK)
    dst0 = dst_p[:e0].reshape(N_SUBCORES, cpt0, CHUNK)
    gidx1 = gidx_p[e0:].reshape(N_SUBCORES, cpt1, CHUNK)
    dst1 = dst_p[e0:].reshape(N_SUBCORES, cpt1, CHUNK)
    zeros_blk = jnp.zeros((stripe, h), jnp.float32)

    wall = param.transpose(1, 0, 2).reshape(h, th)
    k = jnp.arange(th)
    within = k % h
    src_col = (k // h) * h + (within % 2) * (h // 2) + within // 2
    wall = wall[:, src_col]

    sc_agg = _make_sc_agg(n_pad, (cpt0, cpt1), h)

    out = feature
    y = _dense_pre(feature, wall, n, h, th, grid, r)
    for layer in range(3):
        partial = sc_agg(y.reshape(n * t, h), gidx0, dst0, gidx1, dst1,
                         zeros_blk)
        if layer < 2:
            out, y = _dense_mid(out, partial, wall, n, n_pad, h, th, grid, r)
        else:
            probs = _finish(out, partial, W1, b1, W2, b2, W3, b3,
                            n, h, grid, r)
    return probs
